# Initial kernel scaffold; baseline (speedup 1.0000x reference)
import functools

import jax
import jax.numpy as jnp
from jax import lax
from jax.experimental import pallas as pl
from jax.experimental.pallas import tpu as pltpu
from jax.experimental.pallas import tpu_sc as plsc

VOCAB = 1000000
EMBED_DIM = 32
HIDDEN_DIM = 64
NUM_CLASSES = 3
BATCH = 16384
HIST = 50

NC = 2
NS = 16
NW = NC * NS
B_PER_W = BATCH // NW
GROUP = 8
IDS_PER_ROW = 100
ROWS_PER_GROUP = GROUP * HIST
IDROWS_PER_GROUP = ROWS_PER_GROUP // IDS_PER_ROW
NGROUPS = B_PER_W // GROUP


def _sc_gather_pool(ids2d, emb):
    mesh = plsc.VectorSubcoreMesh(core_axis_name="c", subcore_axis_name="s",
                                  num_cores=NC, num_subcores=NS)

    @functools.partial(
        pl.kernel,
        out_type=jax.ShapeDtypeStruct((BATCH, EMBED_DIM), jnp.float32),
        mesh=mesh,
        scratch_types=[
            pltpu.VMEM((IDROWS_PER_GROUP, IDS_PER_ROW), jnp.int32),
            pltpu.VMEM((ROWS_PER_GROUP, EMBED_DIM), jnp.float32),
            pltpu.VMEM((GROUP, EMBED_DIM), jnp.float32),
            pltpu.SemaphoreType.DMA,
        ],
    )
    def k(ids_hbm, emb_hbm, out_hbm, idx_v, rows_v, out_v, sem):
        wid = lax.axis_index("s") * NC + lax.axis_index("c")
        idrow_base = wid * (NGROUPS * IDROWS_PER_GROUP)
        out_base = wid * B_PER_W

        def body(g, carry):
            pltpu.sync_copy(
                ids_hbm.at[pl.ds(idrow_base + g * IDROWS_PER_GROUP,
                                 IDROWS_PER_GROUP)],
                idx_v)
            cps = [
                pltpu.async_copy(
                    emb_hbm.at[idx_v.at[j]],
                    rows_v.at[pl.ds(j * IDS_PER_ROW, IDS_PER_ROW)],
                    sem)
                for j in range(IDROWS_PER_GROUP)
            ]
            for cp in cps:
                cp.wait()
            for b in range(GROUP):
                base = b * HIST
                acc0 = rows_v[base, pl.ds(0, 16)]
                acc1 = rows_v[base, pl.ds(16, 16)]
                for t in range(1, HIST):
                    acc0 = acc0 + rows_v[base + t, pl.ds(0, 16)]
                    acc1 = acc1 + rows_v[base + t, pl.ds(16, 16)]
                out_v[b, pl.ds(0, 16)] = acc0
                out_v[b, pl.ds(16, 16)] = acc1
            pltpu.sync_copy(out_v,
                            out_hbm.at[pl.ds(out_base + g * GROUP, GROUP)])
            return carry

        lax.fori_loop(0, NGROUPS, body, 0)

    return k(ids2d, emb)


def _tc_mlp(pooled, W1, b1, W2, b2):
    tile = 2048
    scale = 1.0 / HIST

    def body(x_ref, w1_ref, b1_ref, w2_ref, b2_ref, o_ref):
        x = x_ref[...] * scale
        h = jnp.dot(x, w1_ref[...], preferred_element_type=jnp.float32)
        h = jnp.maximum(h + b1_ref[...], 0.0)
        o_ref[...] = (jnp.dot(h, w2_ref[...],
                              preferred_element_type=jnp.float32)
                      + b2_ref[...])

    return pl.pallas_call(
        body,
        grid=(BATCH // tile,),
        in_specs=[
            pl.BlockSpec((tile, EMBED_DIM), lambda i: (i, 0)),
            pl.BlockSpec((EMBED_DIM, HIDDEN_DIM), lambda i: (0, 0)),
            pl.BlockSpec((1, HIDDEN_DIM), lambda i: (0, 0)),
            pl.BlockSpec((HIDDEN_DIM, NUM_CLASSES), lambda i: (0, 0)),
            pl.BlockSpec((1, NUM_CLASSES), lambda i: (0, 0)),
        ],
        out_specs=pl.BlockSpec((tile, NUM_CLASSES), lambda i: (i, 0)),
        out_shape=jax.ShapeDtypeStruct((BATCH, NUM_CLASSES), jnp.float32),
    )(pooled, W1, b1.reshape(1, HIDDEN_DIM), W2, b2.reshape(1, NUM_CLASSES))


def kernel(ids, emb, W1, b1, W2, b2):
    ids2d = ids.astype(jnp.int32).reshape(BATCH * HIST // IDS_PER_ROW,
                                          IDS_PER_ROW)
    pooled = _sc_gather_pool(ids2d, emb)
    return _tc_mlp(pooled, W1, b1, W2, b2)


# speedup vs baseline: 2.4403x; 2.4403x over previous
import functools

import jax
import jax.numpy as jnp
from jax import lax
from jax.experimental import pallas as pl
from jax.experimental.pallas import tpu as pltpu
from jax.experimental.pallas import tpu_sc as plsc

VOCAB = 1000000
EMBED_DIM = 32
HIDDEN_DIM = 64
NUM_CLASSES = 3
BATCH = 16384
HIST = 50

NC = 2
NS = 16
NW = NC * NS
B_PER_W = BATCH // NW
GROUP = 8
IDS_PER_ROW = 100
ROWS_PER_GROUP = GROUP * HIST
IDROWS_PER_GROUP = ROWS_PER_GROUP // IDS_PER_ROW
NGROUPS = B_PER_W // GROUP


def _sc_gather_pool(ids2d, emb):
    mesh = plsc.VectorSubcoreMesh(core_axis_name="c", subcore_axis_name="s",
                                  num_cores=NC, num_subcores=NS)

    @functools.partial(
        pl.kernel,
        out_type=jax.ShapeDtypeStruct((BATCH, EMBED_DIM), jnp.float32),
        mesh=mesh,
        scratch_types=[
            pltpu.VMEM((IDROWS_PER_GROUP, IDS_PER_ROW), jnp.int32),
            pltpu.VMEM((ROWS_PER_GROUP, EMBED_DIM), jnp.float32),
            pltpu.VMEM((GROUP, EMBED_DIM), jnp.float32),
            pltpu.SemaphoreType.DMA,
        ],
        compiler_params=pltpu.CompilerParams(use_tc_tiling_on_sc=False),
    )
    def k(ids_hbm, emb_hbm, out_hbm, idx_v, rows_v, out_v, sem):
        wid = lax.axis_index("s") * NC + lax.axis_index("c")
        idrow_base = wid * (NGROUPS * IDROWS_PER_GROUP)
        out_base = wid * B_PER_W

        def body(g, carry):
            pltpu.sync_copy(
                ids_hbm.at[pl.ds(idrow_base + g * IDROWS_PER_GROUP,
                                 IDROWS_PER_GROUP)],
                idx_v)
            cps = [
                pltpu.async_copy(
                    emb_hbm.at[idx_v.at[j]],
                    rows_v.at[pl.ds(j * IDS_PER_ROW, IDS_PER_ROW)],
                    sem)
                for j in range(IDROWS_PER_GROUP)
            ]
            for cp in cps:
                cp.wait()
            for b in range(GROUP):
                base = b * HIST
                acc0 = rows_v[base, pl.ds(0, 16)]
                acc1 = rows_v[base, pl.ds(16, 16)]
                for t in range(1, HIST):
                    acc0 = acc0 + rows_v[base + t, pl.ds(0, 16)]
                    acc1 = acc1 + rows_v[base + t, pl.ds(16, 16)]
                out_v[b, pl.ds(0, 16)] = acc0
                out_v[b, pl.ds(16, 16)] = acc1
            pltpu.sync_copy(out_v,
                            out_hbm.at[pl.ds(out_base + g * GROUP, GROUP)])
            return carry

        lax.fori_loop(0, NGROUPS, body, 0)

    return k(ids2d, emb)


def _tc_mlp(pooled, W1, b1, W2, b2):
    tile = 2048
    scale = 1.0 / HIST

    def body(x_ref, w1_ref, b1_ref, w2_ref, b2_ref, o_ref):
        x = x_ref[...] * scale
        h = jnp.dot(x, w1_ref[...], preferred_element_type=jnp.float32)
        h = jnp.maximum(h + b1_ref[...], 0.0)
        o_ref[...] = (jnp.dot(h, w2_ref[...],
                              preferred_element_type=jnp.float32)
                      + b2_ref[...])

    return pl.pallas_call(
        body,
        grid=(BATCH // tile,),
        in_specs=[
            pl.BlockSpec((tile, EMBED_DIM), lambda i: (i, 0)),
            pl.BlockSpec((EMBED_DIM, HIDDEN_DIM), lambda i: (0, 0)),
            pl.BlockSpec((1, HIDDEN_DIM), lambda i: (0, 0)),
            pl.BlockSpec((HIDDEN_DIM, NUM_CLASSES), lambda i: (0, 0)),
            pl.BlockSpec((1, NUM_CLASSES), lambda i: (0, 0)),
        ],
        out_specs=pl.BlockSpec((tile, NUM_CLASSES), lambda i: (i, 0)),
        out_shape=jax.ShapeDtypeStruct((BATCH, NUM_CLASSES), jnp.float32),
    )(pooled, W1, b1.reshape(1, HIDDEN_DIM), W2, b2.reshape(1, NUM_CLASSES))


def kernel(ids, emb, W1, b1, W2, b2):
    ids2d = ids.astype(jnp.int32).reshape(BATCH * HIST // IDS_PER_ROW,
                                          IDS_PER_ROW)
    pooled = _sc_gather_pool(ids2d, emb)
    return _tc_mlp(pooled, W1, b1, W2, b2)


# speedup vs baseline: 2.5356x; 1.0391x over previous
import functools

import jax
import jax.numpy as jnp
from jax import lax
from jax.experimental import pallas as pl
from jax.experimental.pallas import tpu as pltpu
from jax.experimental.pallas import tpu_sc as plsc

VOCAB = 1000000
EMBED_DIM = 32
HIDDEN_DIM = 64
NUM_CLASSES = 3
BATCH = 16384
HIST = 50

NC = 2
NS = 16
NW = NC * NS
B_PER_W = BATCH // NW
GROUP = 8
IDS_PER_ROW = 100
ROWS_PER_GROUP = GROUP * HIST
IDROWS_PER_GROUP = ROWS_PER_GROUP // IDS_PER_ROW
NGROUPS = B_PER_W // GROUP


def _sc_gather_pool(ids2d, emb):
    mesh = plsc.VectorSubcoreMesh(core_axis_name="c", subcore_axis_name="s",
                                  num_cores=NC, num_subcores=NS)
    idrows_per_w = NGROUPS * IDROWS_PER_GROUP

    @functools.partial(
        pl.kernel,
        out_type=jax.ShapeDtypeStruct((BATCH, EMBED_DIM), jnp.float32),
        mesh=mesh,
        scratch_types=[
            pltpu.VMEM((idrows_per_w, IDS_PER_ROW), jnp.int32),
            pltpu.VMEM((2, ROWS_PER_GROUP, EMBED_DIM), jnp.float32),
            pltpu.VMEM((2, GROUP, EMBED_DIM), jnp.float32),
            pltpu.SemaphoreType.DMA((2,)),
            pltpu.SemaphoreType.DMA((2,)),
        ],
        compiler_params=pltpu.CompilerParams(use_tc_tiling_on_sc=False),
    )
    def k(ids_hbm, emb_hbm, out_hbm, ids_all, rows_v, out_v, gsem, osem):
        wid = lax.axis_index("s") * NC + lax.axis_index("c")
        out_base = wid * B_PER_W

        pltpu.sync_copy(ids_hbm.at[pl.ds(wid * idrows_per_w, idrows_per_w)],
                        ids_all)

        def fire_gathers(s, g):
            for j in range(IDROWS_PER_GROUP):
                pltpu.async_copy(
                    emb_hbm.at[ids_all.at[g * IDROWS_PER_GROUP + j]],
                    rows_v.at[s].at[pl.ds(j * IDS_PER_ROW, IDS_PER_ROW)],
                    gsem.at[s])

        def drain_gathers(s):
            for j in range(IDROWS_PER_GROUP):
                pltpu.make_async_copy(
                    emb_hbm.at[ids_all.at[0]],
                    rows_v.at[s].at[pl.ds(j * IDS_PER_ROW, IDS_PER_ROW)],
                    gsem.at[s]).wait()

        def drain_out(s):
            pltpu.make_async_copy(out_v.at[s],
                                  out_hbm.at[pl.ds(0, GROUP)],
                                  osem.at[s]).wait()

        fire_gathers(0, 0)
        fire_gathers(1, 1)

        def pair_body(i, carry):
            for s in (0, 1):
                g = 2 * i + s
                drain_gathers(s)

                @pl.when(i > 0)
                def _():
                    drain_out(s)

                for b in range(GROUP):
                    base = b * HIST
                    for h in (0, 16):
                        acc = (rows_v[s, base, pl.ds(h, 16)]
                               + rows_v[s, base + HIST - 1, pl.ds(h, 16)])
                        for t in range(1, HIST - 1, 2):
                            pair = (rows_v[s, base + t, pl.ds(h, 16)]
                                    + rows_v[s, base + t + 1, pl.ds(h, 16)])
                            acc = acc + pair
                        out_v[s, b, pl.ds(h, 16)] = acc
                pltpu.async_copy(out_v.at[s],
                                 out_hbm.at[pl.ds(out_base + g * GROUP,
                                                  GROUP)],
                                 osem.at[s])

                @pl.when(g + 2 < NGROUPS)
                def _():
                    fire_gathers(s, g + 2)
            return carry

        lax.fori_loop(0, NGROUPS // 2, pair_body, 0)
        drain_out(0)
        drain_out(1)

    return k(ids2d, emb)


def _tc_mlp(pooled, W1, b1, W2, b2):
    tile = 2048
    scale = 1.0 / HIST

    def body(x_ref, w1_ref, b1_ref, w2_ref, b2_ref, o_ref):
        x = x_ref[...] * scale
        h = jnp.dot(x, w1_ref[...], preferred_element_type=jnp.float32)
        h = jnp.maximum(h + b1_ref[...], 0.0)
        o_ref[...] = (jnp.dot(h, w2_ref[...],
                              preferred_element_type=jnp.float32)
                      + b2_ref[...])

    return pl.pallas_call(
        body,
        grid=(BATCH // tile,),
        in_specs=[
            pl.BlockSpec((tile, EMBED_DIM), lambda i: (i, 0)),
            pl.BlockSpec((EMBED_DIM, HIDDEN_DIM), lambda i: (0, 0)),
            pl.BlockSpec((1, HIDDEN_DIM), lambda i: (0, 0)),
            pl.BlockSpec((HIDDEN_DIM, NUM_CLASSES), lambda i: (0, 0)),
            pl.BlockSpec((1, NUM_CLASSES), lambda i: (0, 0)),
        ],
        out_specs=pl.BlockSpec((tile, NUM_CLASSES), lambda i: (i, 0)),
        out_shape=jax.ShapeDtypeStruct((BATCH, NUM_CLASSES), jnp.float32),
    )(pooled, W1, b1.reshape(1, HIDDEN_DIM), W2, b2.reshape(1, NUM_CLASSES))


def kernel(ids, emb, W1, b1, W2, b2):
    ids2d = ids.astype(jnp.int32).reshape(BATCH * HIST // IDS_PER_ROW,
                                          IDS_PER_ROW)
    pooled = _sc_gather_pool(ids2d, emb)
    return _tc_mlp(pooled, W1, b1, W2, b2)
